# Initial kernel scaffold; baseline (speedup 1.0000x reference)
import jax
import jax.numpy as jnp
from jax.experimental import pallas as pl


def kernel(x, w_scores, b_scores, w_classes, b_classes, w_bboxes, b_bboxes):
    raise NotImplementedError("write your pallas kernel here")


# speedup vs baseline: 1.9684x; 1.9684x over previous
import functools

import jax
import jax.numpy as jnp
from jax.experimental import pallas as pl

B, C, H, W = 16, 96, 128, 128
NCLASSES = 80
NOUT = 1 + NCLASSES + 4
PH, PW = 32.0, 32.0


def _head_kernel(x_ref, w_ref, b_ref, o_ref, *, hw_block, w_cols):
    hw0 = pl.program_id(1) * hw_block
    xb = x_ref[0]
    y = jnp.dot(w_ref[...], xb, preferred_element_type=jnp.float32)
    y = y + b_ref[...]

    idx = jax.lax.broadcasted_iota(jnp.int32, (1, hw_block), 1)
    xx = (idx % w_cols).astype(jnp.float32)
    yy = ((hw0 + idx) // w_cols).astype(jnp.float32)

    r0 = jax.nn.sigmoid(xx * y[NOUT - 4:NOUT - 3])
    r1 = jax.nn.sigmoid(yy * y[NOUT - 3:NOUT - 2])
    r2 = PW * jnp.exp(y[NOUT - 2:NOUT - 1])
    r3 = PH * jnp.exp(y[NOUT - 1:NOUT])

    o_ref[0] = jnp.concatenate([y[: NOUT - 4], r0, r1, r2, r3], axis=0)


def kernel(x, w_scores, b_scores, w_classes, b_classes, w_bboxes, b_bboxes):
    w_all = jnp.concatenate([w_scores, w_classes, w_bboxes], axis=0)
    b_all = jnp.concatenate([b_scores, b_classes, b_bboxes], axis=0)[:, None]

    hw = H * W
    hw_block = 2048
    n_hw = hw // hw_block

    xf = x.reshape(B, C, hw)
    out = pl.pallas_call(
        functools.partial(_head_kernel, hw_block=hw_block, w_cols=W),
        grid=(B, n_hw),
        in_specs=[
            pl.BlockSpec((1, C, hw_block), lambda b, j: (b, 0, j)),
            pl.BlockSpec((NOUT, C), lambda b, j: (0, 0)),
            pl.BlockSpec((NOUT, 1), lambda b, j: (0, 0)),
        ],
        out_specs=pl.BlockSpec((1, NOUT, hw_block), lambda b, j: (b, 0, j)),
        out_shape=jax.ShapeDtypeStruct((B, NOUT, hw), jnp.float32),
    )(xf, w_all, b_all)
    return out.reshape(B, NOUT, H, W)


# speedup vs baseline: 2.3364x; 1.1869x over previous
import functools

import jax
import jax.numpy as jnp
from jax.experimental import pallas as pl
from jax.experimental.pallas import tpu as pltpu

B, C, H, W = 16, 96, 128, 128
NCLASSES = 80
NOUT = 1 + NCLASSES + 4
PH, PW = 32.0, 32.0


def _head_kernel(x_ref, w_ref, b_ref, o_ref, *, hw_block, w_cols):
    hw0 = pl.program_id(1) * hw_block
    xb = x_ref[0]
    y = jnp.dot(w_ref[...], xb, preferred_element_type=jnp.float32)
    y = y + b_ref[...]

    idx = jax.lax.broadcasted_iota(jnp.int32, (1, hw_block), 1)
    xx = (idx % w_cols).astype(jnp.float32)
    yy = ((hw0 + idx) // w_cols).astype(jnp.float32)

    r0 = jax.nn.sigmoid(xx * y[NOUT - 4:NOUT - 3])
    r1 = jax.nn.sigmoid(yy * y[NOUT - 3:NOUT - 2])
    r2 = PW * jnp.exp(y[NOUT - 2:NOUT - 1])
    r3 = PH * jnp.exp(y[NOUT - 1:NOUT])

    o_ref[0] = jnp.concatenate([y[: NOUT - 4], r0, r1, r2, r3], axis=0)


def kernel(x, w_scores, b_scores, w_classes, b_classes, w_bboxes, b_bboxes):
    w_all = jnp.concatenate([w_scores, w_classes, w_bboxes], axis=0)
    b_all = jnp.concatenate([b_scores, b_classes, b_bboxes], axis=0)[:, None]

    hw = H * W
    hw_block = 8192
    n_hw = hw // hw_block

    xf = x.reshape(B, C, hw)
    out = pl.pallas_call(
        functools.partial(_head_kernel, hw_block=hw_block, w_cols=W),
        grid=(B, n_hw),
        in_specs=[
            pl.BlockSpec((1, C, hw_block), lambda b, j: (b, 0, j)),
            pl.BlockSpec((NOUT, C), lambda b, j: (0, 0)),
            pl.BlockSpec((NOUT, 1), lambda b, j: (0, 0)),
        ],
        out_specs=pl.BlockSpec((1, NOUT, hw_block), lambda b, j: (b, 0, j)),
        out_shape=jax.ShapeDtypeStruct((B, NOUT, hw), jnp.float32),
        compiler_params=pltpu.CompilerParams(
            dimension_semantics=("parallel", "parallel"),
        ),
    )(xf, w_all, b_all)
    return out.reshape(B, NOUT, H, W)


# speedup vs baseline: 2.3586x; 1.0095x over previous
import functools

import jax
import jax.numpy as jnp
from jax.experimental import pallas as pl
from jax.experimental.pallas import tpu as pltpu

B, C, H, W = 16, 96, 128, 128
NCLASSES = 80
NOUT = 1 + NCLASSES + 4
PH, PW = 32.0, 32.0


def _head_kernel(x_ref, w_ref, b_ref, o_ref, *, hw_block, w_cols):
    hw0 = pl.program_id(1) * hw_block
    xb = x_ref[0]
    y = jnp.dot(w_ref[...], xb, preferred_element_type=jnp.float32)
    y = y + b_ref[...]

    idx = jax.lax.broadcasted_iota(jnp.int32, (1, hw_block), 1)
    xx = (idx % w_cols).astype(jnp.float32)
    yy = ((hw0 + idx) // w_cols).astype(jnp.float32)

    r0 = jax.nn.sigmoid(xx * y[NOUT - 4:NOUT - 3])
    r1 = jax.nn.sigmoid(yy * y[NOUT - 3:NOUT - 2])
    r2 = PW * jnp.exp(y[NOUT - 2:NOUT - 1])
    r3 = PH * jnp.exp(y[NOUT - 1:NOUT])

    o_ref[0] = jnp.concatenate([y[: NOUT - 4], r0, r1, r2, r3], axis=0)


def kernel(x, w_scores, b_scores, w_classes, b_classes, w_bboxes, b_bboxes):
    w_all = jnp.concatenate([w_scores, w_classes, w_bboxes], axis=0)
    b_all = jnp.concatenate([b_scores, b_classes, b_bboxes], axis=0)[:, None]

    hw = H * W
    hw_block = 16384
    n_hw = hw // hw_block

    xf = x.reshape(B, C, hw)
    out = pl.pallas_call(
        functools.partial(_head_kernel, hw_block=hw_block, w_cols=W),
        grid=(B, n_hw),
        in_specs=[
            pl.BlockSpec((1, C, hw_block), lambda b, j: (b, 0, j)),
            pl.BlockSpec((NOUT, C), lambda b, j: (0, 0)),
            pl.BlockSpec((NOUT, 1), lambda b, j: (0, 0)),
        ],
        out_specs=pl.BlockSpec((1, NOUT, hw_block), lambda b, j: (b, 0, j)),
        out_shape=jax.ShapeDtypeStruct((B, NOUT, hw), jnp.float32),
        compiler_params=pltpu.CompilerParams(
            dimension_semantics=("parallel", "parallel"),
        ),
    )(xf, w_all, b_all)
    return out.reshape(B, NOUT, H, W)


# speedup vs baseline: 2.3648x; 1.0026x over previous
import functools

import jax
import jax.numpy as jnp
from jax.experimental import pallas as pl
from jax.experimental.pallas import tpu as pltpu

B, C, H, W = 16, 96, 128, 128
NCLASSES = 80
NOUT = 1 + NCLASSES + 4
PH, PW = 32.0, 32.0


def _head_kernel(x_ref, w_ref, b_ref, o_ref, *, nb, w_cols):
    hw = x_ref.shape[2]
    idx = jax.lax.broadcasted_iota(jnp.int32, (1, hw), 1)
    xx = (idx % w_cols).astype(jnp.float32)
    yy = (idx // w_cols).astype(jnp.float32)
    for i in range(nb):
        y = jnp.dot(w_ref[...], x_ref[i], preferred_element_type=jnp.float32)
        y = y + b_ref[...]
        r0 = jax.nn.sigmoid(xx * y[NOUT - 4:NOUT - 3])
        r1 = jax.nn.sigmoid(yy * y[NOUT - 3:NOUT - 2])
        r2 = PW * jnp.exp(y[NOUT - 2:NOUT - 1])
        r3 = PH * jnp.exp(y[NOUT - 1:NOUT])
        o_ref[i] = jnp.concatenate([y[: NOUT - 4], r0, r1, r2, r3], axis=0)


def kernel(x, w_scores, b_scores, w_classes, b_classes, w_bboxes, b_bboxes):
    w_all = jnp.concatenate([w_scores, w_classes, w_bboxes], axis=0)
    b_all = jnp.concatenate([b_scores, b_classes, b_bboxes], axis=0)[:, None]

    hw = H * W
    nb = 2

    xf = x.reshape(B, C, hw)
    out = pl.pallas_call(
        functools.partial(_head_kernel, nb=nb, w_cols=W),
        grid=(B // nb,),
        in_specs=[
            pl.BlockSpec((nb, C, hw), lambda b: (b, 0, 0)),
            pl.BlockSpec((NOUT, C), lambda b: (0, 0)),
            pl.BlockSpec((NOUT, 1), lambda b: (0, 0)),
        ],
        out_specs=pl.BlockSpec((nb, NOUT, hw), lambda b: (b, 0, 0)),
        out_shape=jax.ShapeDtypeStruct((B, NOUT, hw), jnp.float32),
        compiler_params=pltpu.CompilerParams(
            dimension_semantics=("parallel",),
        ),
    )(xf, w_all, b_all)
    return out.reshape(B, NOUT, H, W)
